# baseline (speedup 1.0000x reference)
import functools

import jax
import jax.numpy as jnp
from jax import lax
from jax.experimental import pallas as pl
from jax.experimental.pallas import tpu as pltpu
from jax.experimental.pallas import tpu_sc as plsc

VOCAB = 1000
VPAD = 1024
W32 = VPAD // 2
NQ = W32 // 128
D_MODEL = 64
BATCH = 1024
SEQ = 50
TPAD = 56

_info = plsc.get_sparse_core_info()
NC, NS = _info.num_cores, _info.num_subcores
NW = NC * NS
B_PER_W = BATCH // NW
IDX_PER_W = B_PER_W * NQ * TPAD


def _mm_body(a_ref, b_ref, o_ref):
    o_ref[...] = lax.dot_general(
        a_ref[...], b_ref[...],
        (((1,), (1,)), ((), ())),
        preferred_element_type=jnp.float32,
    ).astype(jnp.bfloat16)


def _token_logit_table(embed_table, head_w_pad):
    return pl.pallas_call(
        _mm_body,
        out_shape=jax.ShapeDtypeStruct((VOCAB, VPAD), jnp.bfloat16),
    )(embed_table, head_w_pad)


_mesh = plsc.VectorSubcoreMesh(core_axis_name="c", subcore_axis_name="s")


@functools.partial(
    pl.kernel,
    mesh=_mesh,
    out_type=jax.ShapeDtypeStruct((BATCH, SEQ, W32), jnp.int32),
    scratch_types=[
        pltpu.VMEM((IDX_PER_W,), jnp.int32),
        pltpu.VMEM((SEQ, W32), jnp.int32),
        pltpu.VMEM((SEQ, W32), jnp.int32),
        pltpu.SemaphoreType.DMA,
        pltpu.SemaphoreType.DMA,
    ],
)
def _gather_rows(m32_hbm, idx_hbm, out_hbm, idx_v, buf0, buf1, sem0, sem1):
    wid = lax.axis_index("s") * NC + lax.axis_index("c")
    pltpu.sync_copy(idx_hbm.at[pl.ds(wid * IDX_PER_W, IDX_PER_W)], idx_v)

    def copies(j, buf, sem):
        return [
            pltpu.make_async_copy(
                m32_hbm.at[idx_v.at[pl.ds((j * NQ + q) * TPAD, SEQ)]],
                buf.at[:, pl.ds(128 * q, 128)],
                sem,
            )
            for q in range(NQ)
        ]

    def start(j, buf, sem):
        for c in copies(j, buf, sem):
            c.start()

    def finish(j, buf, sem):
        for c in copies(j, buf, sem):
            c.wait()
        pltpu.sync_copy(buf, out_hbm.at[wid * B_PER_W + j])

    start(0, buf0, sem0)

    def body(g, carry):
        j0 = 2 * g
        start(j0 + 1, buf1, sem1)
        finish(j0, buf0, sem0)

        @pl.when(j0 + 2 < B_PER_W)
        def _():
            start(j0 + 2, buf0, sem0)

        finish(j0 + 1, buf1, sem1)
        return carry

    lax.fori_loop(0, B_PER_W // 2, body, 0)


def kernel(input_ids, embed_table, head_w):
    head_pad = jnp.pad(head_w, ((0, VPAD - VOCAB), (0, 0)))
    m = _token_logit_table(embed_table, head_pad)
    m32 = lax.bitcast_convert_type(
        m.reshape(VOCAB, W32, 2), jnp.int32
    ).reshape(VOCAB * NQ, 128)
    ids = input_ids.astype(jnp.int32)
    idx_all = (NQ * ids)[:, None, :] + jnp.arange(NQ, dtype=jnp.int32)[None, :, None]
    idx_all = jnp.pad(idx_all, ((0, 0), (0, 0), (0, TPAD - SEQ)))
    out32 = _gather_rows(m32, idx_all.reshape(-1))
    out_bf = lax.bitcast_convert_type(out32, jnp.bfloat16).reshape(BATCH, SEQ, VPAD)
    return out_bf[:, :, :VOCAB].astype(jnp.float32)


# speedup vs baseline: 2.2194x; 2.2194x over previous
import functools

import jax
import jax.numpy as jnp
from jax import lax
from jax.experimental import pallas as pl
from jax.experimental.pallas import tpu as pltpu
from jax.experimental.pallas import tpu_sc as plsc

VOCAB = 1000
VPAD = 1024
NTC = VPAD // 128
D_MODEL = 64
BATCH = 1024
SEQ = 50
TPAD = 56

_info = plsc.get_sparse_core_info()
NC, NS = _info.num_cores, _info.num_subcores
NW = NC * NS
B_PER_W = BATCH // NW
IDX_PER_W = B_PER_W * NTC * TPAD


def _mm_body(a_ref, b_ref, o_ref):
    o_ref[...] = lax.dot_general(
        a_ref[...], b_ref[...],
        (((1,), (1,)), ((), ())),
        preferred_element_type=jnp.float32,
    )


def _token_logit_table(embed_table, head_w_pad):
    return pl.pallas_call(
        _mm_body,
        out_shape=jax.ShapeDtypeStruct((VOCAB, VPAD), jnp.float32),
    )(embed_table, head_w_pad)


_mesh = plsc.VectorSubcoreMesh(core_axis_name="c", subcore_axis_name="s")


@functools.partial(
    pl.kernel,
    mesh=_mesh,
    out_type=jax.ShapeDtypeStruct((BATCH, SEQ, VPAD), jnp.float32),
    scratch_types=[
        pltpu.VMEM((IDX_PER_W,), jnp.int32),
        pltpu.VMEM((SEQ, VPAD), jnp.float32),
        pltpu.VMEM((SEQ, VPAD), jnp.float32),
        pltpu.SemaphoreType.DMA,
        pltpu.SemaphoreType.DMA,
    ],
)
def _gather_rows(m8_hbm, idx_hbm, out_hbm, idx_v, buf0, buf1, sem0, sem1):
    wid = lax.axis_index("s") * NC + lax.axis_index("c")
    pltpu.sync_copy(idx_hbm.at[pl.ds(wid * IDX_PER_W, IDX_PER_W)], idx_v)

    def copies(j, buf, sem):
        return [
            pltpu.make_async_copy(
                m8_hbm.at[idx_v.at[pl.ds((j * NTC + tc) * TPAD, SEQ)]],
                buf.at[:, pl.ds(128 * tc, 128)],
                sem,
            )
            for tc in range(NTC)
        ]

    def start(j, buf, sem):
        for c in copies(j, buf, sem):
            c.start()

    def finish(j, buf, sem):
        for c in copies(j, buf, sem):
            c.wait()
        pltpu.sync_copy(buf, out_hbm.at[wid * B_PER_W + j])

    start(0, buf0, sem0)

    def body(g, carry):
        j0 = 2 * g
        start(j0 + 1, buf1, sem1)
        finish(j0, buf0, sem0)

        @pl.when(j0 + 2 < B_PER_W)
        def _():
            start(j0 + 2, buf0, sem0)

        finish(j0 + 1, buf1, sem1)
        return carry

    lax.fori_loop(0, B_PER_W // 2, body, 0)


def kernel(input_ids, embed_table, head_w):
    head_pad = jnp.pad(head_w, ((0, VPAD - VOCAB), (0, 0)))
    m = _token_logit_table(embed_table, head_pad)
    m8 = m.reshape(VOCAB * NTC, 128)
    ids = input_ids.astype(jnp.int32)
    idx_all = (NTC * ids)[:, None, :] + jnp.arange(NTC, dtype=jnp.int32)[None, :, None]
    idx_all = jnp.pad(idx_all, ((0, 0), (0, 0), (0, TPAD - SEQ)))
    return _gather_rows(m8, idx_all.reshape(-1))[:, :, :VOCAB]
